# baseline (speedup 1.0000x reference)
import jax
import jax.numpy as jnp
from jax import lax
from jax.experimental import pallas as pl
from jax.experimental.pallas import tpu as pltpu
from jax.experimental.pallas import tpu_sc as plsc

N_NODES = 10000
N_EDGES = 320000
D = 128

NC = 2
NS = 16
NW = NC * NS
EPW = N_EDGES // NW
CHUNK = 128
NCHUNK = EPW // CHUNK
GTAIL = EPW - NCHUNK * CHUNK
SCHUNK = 128
NSC = EPW // SCHUNK
TAIL = EPW - NSC * SCHUNK
N_PAD = 10240
ROWS_PER_TILE = N_PAD // NS

_f32 = jnp.float32


def _proj_body(nf_ref, ws_ref, wd_ref, ps_ref, pd_ref):
    nf = nf_ref[...]
    ps_ref[...] = jnp.dot(nf, ws_ref[...], preferred_element_type=_f32)
    pd_ref[...] = jnp.dot(nf, wd_ref[...], preferred_element_type=_f32)


def _node_proj(nf, ws, wd):
    blk = 2000
    return pl.pallas_call(
        _proj_body,
        grid=(N_NODES // blk,),
        in_specs=[
            pl.BlockSpec((blk, D), lambda i: (i, 0)),
            pl.BlockSpec((D, D), lambda i: (0, 0)),
            pl.BlockSpec((D, D), lambda i: (0, 0)),
        ],
        out_specs=[
            pl.BlockSpec((blk, D), lambda i: (i, 0)),
            pl.BlockSpec((blk, D), lambda i: (i, 0)),
        ],
        out_shape=[jax.ShapeDtypeStruct((N_NODES, D), _f32)] * 2,
    )(nf, ws, wd)


def _edge_body(ef_ref, gsum_ref, w0_ref, b0_ref, w1_ref, b1_ref,
               w2_ref, b2_ref, out_ref):
    ef = ef_ref[...]
    h = jnp.dot(ef, w0_ref[...], preferred_element_type=_f32)
    h = jnp.maximum(h + gsum_ref[...] + b0_ref[...], 0.0)
    h = jnp.maximum(
        jnp.dot(h, w1_ref[...], preferred_element_type=_f32) + b1_ref[...], 0.0)
    out_ref[...] = ef + jnp.dot(h, w2_ref[...], preferred_element_type=_f32) \
        + b2_ref[...]


def _edge_mlp(ef, gsum, w0, b0, w1, b1, w2, b2):
    blk = 2000
    wspec = pl.BlockSpec((D, D), lambda i: (0, 0))
    bspec = pl.BlockSpec((1, D), lambda i: (0, 0))
    espec = pl.BlockSpec((blk, D), lambda i: (i, 0))
    return pl.pallas_call(
        _edge_body,
        grid=(N_EDGES // blk,),
        in_specs=[espec, espec, wspec, bspec, wspec, bspec, wspec, bspec],
        out_specs=espec,
        out_shape=jax.ShapeDtypeStruct((N_EDGES, D), _f32),
    )(ef, gsum, w0, b0.reshape(1, D), w1, b1.reshape(1, D), w2,
      b2.reshape(1, D))


def _node_body(nf_ref, p_ref, w0n_ref, w0a_ref, b0_ref, w1_ref, b1_ref,
               w2_ref, b2_ref, out_ref):
    nf = nf_ref[...]
    agg = p_ref[0] + p_ref[1]
    h = jnp.dot(nf, w0n_ref[...], preferred_element_type=_f32) \
        + jnp.dot(agg, w0a_ref[...], preferred_element_type=_f32)
    h = jnp.maximum(h + b0_ref[...], 0.0)
    h = jnp.maximum(
        jnp.dot(h, w1_ref[...], preferred_element_type=_f32) + b1_ref[...], 0.0)
    out_ref[...] = nf + jnp.dot(h, w2_ref[...], preferred_element_type=_f32) \
        + b2_ref[...]


def _node_mlp(nf, partials, w0n, w0a, b0, w1, b1, w2, b2):
    blk = 2000
    wspec = pl.BlockSpec((D, D), lambda i: (0, 0))
    bspec = pl.BlockSpec((1, D), lambda i: (0, 0))
    nspec = pl.BlockSpec((blk, D), lambda i: (i, 0))
    return pl.pallas_call(
        _node_body,
        grid=(N_NODES // blk,),
        in_specs=[
            nspec,
            pl.BlockSpec((NC, blk, D), lambda i: (0, i, 0)),
            wspec, wspec, bspec, wspec, bspec, wspec, bspec,
        ],
        out_specs=nspec,
        out_shape=jax.ShapeDtypeStruct((N_NODES, D), _f32),
    )(nf, partials, w0n, w0a, b0.reshape(1, D), w1, b1.reshape(1, D), w2,
      b2.reshape(1, D))


def _sc_gather_body(ps_hbm, pd_hbm, src_hbm, dst_hbm, gsum_hbm,
                    idx_sv, idx_dv, rows,
                    sem_a0, sem_a1, sem_a2, sem_a3, sem_a4, sem_a5,
                    sem_b0, sem_b1, sem_b2, sem_b3, sem_b4, sem_b5):
    c = lax.axis_index("c")
    s = lax.axis_index("s")
    wid = s * NC + c
    ebase = wid * EPW
    sem_a = (sem_a0, sem_a1, sem_a2, sem_a3, sem_a4, sem_a5)
    sem_b = (sem_b0, sem_b1, sem_b2, sem_b3, sem_b4, sem_b5)

    pltpu.sync_copy(src_hbm.at[pl.ds(ebase, EPW)], idx_sv)
    pltpu.sync_copy(dst_hbm.at[pl.ds(ebase, EPW)], idx_dv)

    def issue_a(j, p):
        ia = idx_sv.at[pl.ds(j * CHUNK, CHUNK)]
        pltpu.async_copy(ps_hbm.at[ia], rows.at[p], sem_a[p])

    def wait_a(p):
        ia = idx_sv.at[pl.ds(0, CHUNK)]
        pltpu.make_async_copy(ps_hbm.at[ia], rows.at[p], sem_a[p]).wait()

    def issue_b(j, p):
        ib = idx_dv.at[pl.ds(j * CHUNK, CHUNK)]
        pltpu.async_copy(pd_hbm.at[ib], rows.at[p], sem_b[p], add=True)

    def wait_b(p):
        ib = idx_dv.at[pl.ds(0, CHUNK)]
        pltpu.make_async_copy(pd_hbm.at[ib], rows.at[p], sem_b[p]).wait()

    def write(j, p):
        base = ebase + j * CHUNK
        pltpu.sync_copy(rows.at[p], gsum_hbm.at[pl.ds(base, CHUNK)])

    def full_step(j, s, s2, guard_refill):
        wait_a(s)
        issue_b(j, s)
        wait_b(s2)
        write(j - 2, s2)
        if guard_refill:
            @pl.when(j + 4 < NCHUNK)
            def _():
                issue_a(j + 4, s2)
        else:
            issue_a(j + 4, s2)

    issue_a(0, 0)
    issue_a(1, 1)
    issue_a(2, 2)
    issue_a(3, 3)
    wait_a(0)
    issue_b(0, 0)
    issue_a(4, 4)
    wait_a(1)
    issue_b(1, 1)
    issue_a(5, 5)

    def sextet(g, carry):
        j0 = 6 * g + 2
        full_step(j0, 2, 0, False)
        full_step(j0 + 1, 3, 1, False)
        full_step(j0 + 2, 4, 2, False)
        full_step(j0 + 3, 5, 3, False)
        full_step(j0 + 4, 0, 4, False)
        full_step(j0 + 5, 1, 5, True)
        return carry

    lax.fori_loop(0, (NCHUNK - 6) // 6, sextet, 0)
    wait_a(2)
    issue_b(NCHUNK - 4, 2)
    wait_b(0)
    write(NCHUNK - 6, 0)
    wait_a(3)
    issue_b(NCHUNK - 3, 3)
    wait_b(1)
    write(NCHUNK - 5, 1)
    wait_a(4)
    issue_b(NCHUNK - 2, 4)
    wait_b(2)
    write(NCHUNK - 4, 2)
    wait_a(5)
    issue_b(NCHUNK - 1, 5)
    wait_b(3)
    write(NCHUNK - 3, 3)
    wait_b(4)
    write(NCHUNK - 2, 4)
    wait_b(5)
    write(NCHUNK - 1, 5)
    tbase = NCHUNK * CHUNK
    ia_t = idx_sv.at[pl.ds(tbase, GTAIL)]
    ib_t = idx_dv.at[pl.ds(tbase, GTAIL)]
    pltpu.sync_copy(ps_hbm.at[ia_t], rows.at[0, pl.ds(0, GTAIL)])
    pltpu.sync_copy(pd_hbm.at[ib_t], rows.at[0, pl.ds(0, GTAIL)], add=True)
    pltpu.sync_copy(rows.at[0, pl.ds(0, GTAIL)],
                    gsum_hbm.at[pl.ds(ebase + tbase, GTAIL)])


def _sc_gather(ps, pd, src, dst):
    f = pl.kernel(
        _sc_gather_body,
        out_type=jax.ShapeDtypeStruct((N_EDGES, D), _f32),
        mesh=plsc.VectorSubcoreMesh(core_axis_name="c", subcore_axis_name="s",
                                    num_cores=NC, num_subcores=NS),
        scratch_types=[
            pltpu.VMEM((EPW,), jnp.int32),
            pltpu.VMEM((EPW,), jnp.int32),
            pltpu.VMEM((6, CHUNK, D), _f32),
        ] + [pltpu.SemaphoreType.DMA] * 12,
    )
    return f(ps, pd, src, dst)


def _sc_scatter_body(ne_hbm, dst_hbm, zeros_hbm, out_hbm,
                     acc_shared, idx_v, rows_v, sem_l0, sem_l1):
    c = lax.axis_index("c")
    s = lax.axis_index("s")
    wid = s * NC + c
    ebase = wid * EPW
    row0 = s * ROWS_PER_TILE
    sem_l = (sem_l0, sem_l1)
    pltpu.sync_copy(zeros_hbm.at[pl.ds(row0, ROWS_PER_TILE)],
                    acc_shared.at[pl.ds(row0, ROWS_PER_TILE)])
    pltpu.sync_copy(dst_hbm.at[pl.ds(ebase, EPW)], idx_v)
    plsc.subcore_barrier()

    def load(j, p):
        base = ebase + j * SCHUNK
        pltpu.async_copy(ne_hbm.at[pl.ds(base, SCHUNK)], rows_v.at[p],
                         sem_l[p])

    def wait_load(p):
        pltpu.make_async_copy(ne_hbm.at[pl.ds(0, SCHUNK)], rows_v.at[p],
                              sem_l[p]).wait()

    def scat(j, p):
        idx = idx_v.at[pl.ds(j * SCHUNK, SCHUNK)]
        pltpu.sync_copy(rows_v.at[p], acc_shared.at[idx], add=True)

    load(0, 0)
    load(1, 1)

    def pair(jj, carry):
        e = 2 * jj
        o = e + 1
        wait_load(0)
        scat(e, 0)

        @pl.when(e + 2 < NSC)
        def _():
            load(e + 2, 0)

        wait_load(1)
        scat(o, 1)

        @pl.when(o + 2 < NSC)
        def _():
            load(o + 2, 1)

        return carry

    lax.fori_loop(0, NSC // 2, pair, 0)
    pltpu.sync_copy(ne_hbm.at[pl.ds(ebase + NSC * SCHUNK, TAIL)],
                    rows_v.at[0, pl.ds(0, TAIL)])
    pltpu.sync_copy(rows_v.at[0, pl.ds(0, TAIL)],
                    acc_shared.at[idx_v.at[pl.ds(NSC * SCHUNK, TAIL)]],
                    add=True)

    plsc.subcore_barrier()
    pltpu.sync_copy(acc_shared.at[pl.ds(row0, ROWS_PER_TILE)],
                    out_hbm.at[c, pl.ds(row0, ROWS_PER_TILE)])


def _sc_scatter(ne, dst, zeros):
    f = pl.kernel(
        _sc_scatter_body,
        out_type=jax.ShapeDtypeStruct((NC, N_PAD, D), _f32),
        mesh=plsc.VectorSubcoreMesh(core_axis_name="c", subcore_axis_name="s",
                                    num_cores=NC, num_subcores=NS),
        scratch_types=[
            pltpu.VMEM_SHARED((N_PAD, D), _f32),
            pltpu.VMEM((EPW,), jnp.int32),
            pltpu.VMEM((2, SCHUNK, D), _f32),
            pltpu.SemaphoreType.DMA,
            pltpu.SemaphoreType.DMA,
        ],
    )
    return f(ne, dst, zeros)


def kernel(node_feat, edge_feat, edge_index,
           ew0, eb0, ew1, eb1, ew2, eb2,
           nw0, nb0, nw1, nb1, nw2, nb2):
    src = edge_index[0].astype(jnp.int32)
    dst = edge_index[1].astype(jnp.int32)
    ps, pd = _node_proj(node_feat, ew0[D:2 * D], ew0[2 * D:])
    gsum = _sc_gather(ps, pd, src, dst)
    new_edge = _edge_mlp(edge_feat, gsum, ew0[:D], eb0, ew1, eb1, ew2, eb2)
    partials = _sc_scatter(new_edge, dst, jnp.zeros((N_PAD, D), _f32))
    new_node = _node_mlp(node_feat, partials, nw0[:D], nw0[D:], nb0,
                         nw1, nb1, nw2, nb2)
    return (new_node, new_edge)


# speedup vs baseline: 1.0064x; 1.0064x over previous
import jax
import jax.numpy as jnp
from jax import lax
from jax.experimental import pallas as pl
from jax.experimental.pallas import tpu as pltpu
from jax.experimental.pallas import tpu_sc as plsc

N_NODES = 10000
N_EDGES = 320000
D = 128

NC = 2
NS = 16
NW = NC * NS
EPW = N_EDGES // NW
CHUNK = 80
NCHUNK = EPW // CHUNK
SCHUNK = 128
NSC = EPW // SCHUNK
TAIL = EPW - NSC * SCHUNK
N_PAD = 10240
ROWS_PER_TILE = N_PAD // NS

_f32 = jnp.float32


def _proj_body(nf_ref, ws_ref, wd_ref, ps_ref, pd_ref):
    nf = nf_ref[...]
    ps_ref[...] = jnp.dot(nf, ws_ref[...], preferred_element_type=_f32)
    pd_ref[...] = jnp.dot(nf, wd_ref[...], preferred_element_type=_f32)


def _node_proj(nf, ws, wd):
    blk = 2000
    return pl.pallas_call(
        _proj_body,
        grid=(N_NODES // blk,),
        in_specs=[
            pl.BlockSpec((blk, D), lambda i: (i, 0)),
            pl.BlockSpec((D, D), lambda i: (0, 0)),
            pl.BlockSpec((D, D), lambda i: (0, 0)),
        ],
        out_specs=[
            pl.BlockSpec((blk, D), lambda i: (i, 0)),
            pl.BlockSpec((blk, D), lambda i: (i, 0)),
        ],
        out_shape=[jax.ShapeDtypeStruct((N_NODES, D), _f32)] * 2,
    )(nf, ws, wd)


def _edge_body(ef_ref, gsum_ref, w0_ref, b0_ref, w1_ref, b1_ref,
               w2_ref, b2_ref, out_ref):
    ef = ef_ref[...]
    h = jnp.dot(ef, w0_ref[...], preferred_element_type=_f32)
    h = jnp.maximum(h + gsum_ref[...] + b0_ref[...], 0.0)
    h = jnp.maximum(
        jnp.dot(h, w1_ref[...], preferred_element_type=_f32) + b1_ref[...], 0.0)
    out_ref[...] = ef + jnp.dot(h, w2_ref[...], preferred_element_type=_f32) \
        + b2_ref[...]


def _edge_mlp(ef, gsum, w0, b0, w1, b1, w2, b2):
    blk = 2000
    wspec = pl.BlockSpec((D, D), lambda i: (0, 0))
    bspec = pl.BlockSpec((1, D), lambda i: (0, 0))
    espec = pl.BlockSpec((blk, D), lambda i: (i, 0))
    return pl.pallas_call(
        _edge_body,
        grid=(N_EDGES // blk,),
        in_specs=[espec, espec, wspec, bspec, wspec, bspec, wspec, bspec],
        out_specs=espec,
        out_shape=jax.ShapeDtypeStruct((N_EDGES, D), _f32),
    )(ef, gsum, w0, b0.reshape(1, D), w1, b1.reshape(1, D), w2,
      b2.reshape(1, D))


def _node_body(nf_ref, p_ref, w0n_ref, w0a_ref, b0_ref, w1_ref, b1_ref,
               w2_ref, b2_ref, out_ref):
    nf = nf_ref[...]
    agg = p_ref[0] + p_ref[1]
    h = jnp.dot(nf, w0n_ref[...], preferred_element_type=_f32) \
        + jnp.dot(agg, w0a_ref[...], preferred_element_type=_f32)
    h = jnp.maximum(h + b0_ref[...], 0.0)
    h = jnp.maximum(
        jnp.dot(h, w1_ref[...], preferred_element_type=_f32) + b1_ref[...], 0.0)
    out_ref[...] = nf + jnp.dot(h, w2_ref[...], preferred_element_type=_f32) \
        + b2_ref[...]


def _node_mlp(nf, partials, w0n, w0a, b0, w1, b1, w2, b2):
    blk = 2000
    wspec = pl.BlockSpec((D, D), lambda i: (0, 0))
    bspec = pl.BlockSpec((1, D), lambda i: (0, 0))
    nspec = pl.BlockSpec((blk, D), lambda i: (i, 0))
    return pl.pallas_call(
        _node_body,
        grid=(N_NODES // blk,),
        in_specs=[
            nspec,
            pl.BlockSpec((NC, blk, D), lambda i: (0, i, 0)),
            wspec, wspec, bspec, wspec, bspec, wspec, bspec,
        ],
        out_specs=nspec,
        out_shape=jax.ShapeDtypeStruct((N_NODES, D), _f32),
    )(nf, partials, w0n, w0a, b0.reshape(1, D), w1, b1.reshape(1, D), w2,
      b2.reshape(1, D))


def _sc_gather_body(ps_hbm, pd_hbm, src_hbm, dst_hbm, gsum_hbm,
                    idx_sv, idx_dv, rows,
                    sem_a0, sem_a1, sem_a2, sem_a3, sem_a4, sem_a5,
                    sem_b0, sem_b1, sem_b2, sem_b3, sem_b4, sem_b5):
    c = lax.axis_index("c")
    s = lax.axis_index("s")
    wid = s * NC + c
    ebase = wid * EPW
    sem_a = (sem_a0, sem_a1, sem_a2, sem_a3, sem_a4, sem_a5)
    sem_b = (sem_b0, sem_b1, sem_b2, sem_b3, sem_b4, sem_b5)

    pltpu.sync_copy(src_hbm.at[pl.ds(ebase, EPW)], idx_sv)
    pltpu.sync_copy(dst_hbm.at[pl.ds(ebase, EPW)], idx_dv)

    def issue_a(j, p):
        ia = idx_sv.at[pl.ds(j * CHUNK, CHUNK)]
        pltpu.async_copy(ps_hbm.at[ia], rows.at[p], sem_a[p])

    def wait_a(p):
        ia = idx_sv.at[pl.ds(0, CHUNK)]
        pltpu.make_async_copy(ps_hbm.at[ia], rows.at[p], sem_a[p]).wait()

    def issue_b(j, p):
        ib = idx_dv.at[pl.ds(j * CHUNK, CHUNK)]
        pltpu.async_copy(pd_hbm.at[ib], rows.at[p], sem_b[p], add=True)

    def wait_b(p):
        ib = idx_dv.at[pl.ds(0, CHUNK)]
        pltpu.make_async_copy(pd_hbm.at[ib], rows.at[p], sem_b[p]).wait()

    def write(j, p):
        base = ebase + j * CHUNK
        pltpu.sync_copy(rows.at[p], gsum_hbm.at[pl.ds(base, CHUNK)])

    def full_step(j, s, s2, guard_refill):
        wait_a(s)
        issue_b(j, s)
        wait_b(s2)
        write(j - 2, s2)
        if guard_refill:
            @pl.when(j + 4 < NCHUNK)
            def _():
                issue_a(j + 4, s2)
        else:
            issue_a(j + 4, s2)

    issue_a(0, 0)
    issue_a(1, 1)
    issue_a(2, 2)
    issue_a(3, 3)
    wait_a(0)
    issue_b(0, 0)
    issue_a(4, 4)
    wait_a(1)
    issue_b(1, 1)
    issue_a(5, 5)

    def sextet(g, carry):
        j0 = 6 * g + 2
        full_step(j0, 2, 0, False)
        full_step(j0 + 1, 3, 1, False)
        full_step(j0 + 2, 4, 2, False)
        full_step(j0 + 3, 5, 3, False)
        full_step(j0 + 4, 0, 4, False)
        full_step(j0 + 5, 1, 5, True)
        return carry

    lax.fori_loop(0, (NCHUNK - 5) // 6, sextet, 0)
    wait_a(2)
    issue_b(NCHUNK - 3, 2)
    wait_b(0)
    write(NCHUNK - 5, 0)
    wait_a(3)
    issue_b(NCHUNK - 2, 3)
    wait_b(1)
    write(NCHUNK - 4, 1)
    wait_a(4)
    issue_b(NCHUNK - 1, 4)
    wait_b(2)
    write(NCHUNK - 3, 2)
    wait_b(3)
    write(NCHUNK - 2, 3)
    wait_b(4)
    write(NCHUNK - 1, 4)


def _sc_gather(ps, pd, src, dst):
    f = pl.kernel(
        _sc_gather_body,
        out_type=jax.ShapeDtypeStruct((N_EDGES, D), _f32),
        mesh=plsc.VectorSubcoreMesh(core_axis_name="c", subcore_axis_name="s",
                                    num_cores=NC, num_subcores=NS),
        scratch_types=[
            pltpu.VMEM((EPW,), jnp.int32),
            pltpu.VMEM((EPW,), jnp.int32),
            pltpu.VMEM((6, CHUNK, D), _f32),
        ] + [pltpu.SemaphoreType.DMA] * 12,
    )
    return f(ps, pd, src, dst)


def _sc_scatter_body(ne_hbm, dst_hbm, zeros_hbm, out_hbm,
                     acc_shared, idx_v, rows_v, sem_l0, sem_l1):
    c = lax.axis_index("c")
    s = lax.axis_index("s")
    wid = s * NC + c
    ebase = wid * EPW
    row0 = s * ROWS_PER_TILE
    sem_l = (sem_l0, sem_l1)
    pltpu.sync_copy(zeros_hbm.at[pl.ds(row0, ROWS_PER_TILE)],
                    acc_shared.at[pl.ds(row0, ROWS_PER_TILE)])
    pltpu.sync_copy(dst_hbm.at[pl.ds(ebase, EPW)], idx_v)
    plsc.subcore_barrier()

    def load(j, p):
        base = ebase + j * SCHUNK
        pltpu.async_copy(ne_hbm.at[pl.ds(base, SCHUNK)], rows_v.at[p],
                         sem_l[p])

    def wait_load(p):
        pltpu.make_async_copy(ne_hbm.at[pl.ds(0, SCHUNK)], rows_v.at[p],
                              sem_l[p]).wait()

    def scat(j, p):
        idx = idx_v.at[pl.ds(j * SCHUNK, SCHUNK)]
        pltpu.sync_copy(rows_v.at[p], acc_shared.at[idx], add=True)

    load(0, 0)
    load(1, 1)

    def pair(jj, carry):
        e = 2 * jj
        o = e + 1
        wait_load(0)
        scat(e, 0)

        @pl.when(e + 2 < NSC)
        def _():
            load(e + 2, 0)

        wait_load(1)
        scat(o, 1)

        @pl.when(o + 2 < NSC)
        def _():
            load(o + 2, 1)

        return carry

    lax.fori_loop(0, NSC // 2, pair, 0)
    pltpu.sync_copy(ne_hbm.at[pl.ds(ebase + NSC * SCHUNK, TAIL)],
                    rows_v.at[0, pl.ds(0, TAIL)])
    pltpu.sync_copy(rows_v.at[0, pl.ds(0, TAIL)],
                    acc_shared.at[idx_v.at[pl.ds(NSC * SCHUNK, TAIL)]],
                    add=True)

    plsc.subcore_barrier()
    pltpu.sync_copy(acc_shared.at[pl.ds(row0, ROWS_PER_TILE)],
                    out_hbm.at[c, pl.ds(row0, ROWS_PER_TILE)])


def _sc_scatter(ne, dst, zeros):
    f = pl.kernel(
        _sc_scatter_body,
        out_type=jax.ShapeDtypeStruct((NC, N_PAD, D), _f32),
        mesh=plsc.VectorSubcoreMesh(core_axis_name="c", subcore_axis_name="s",
                                    num_cores=NC, num_subcores=NS),
        scratch_types=[
            pltpu.VMEM_SHARED((N_PAD, D), _f32),
            pltpu.VMEM((EPW,), jnp.int32),
            pltpu.VMEM((2, SCHUNK, D), _f32),
            pltpu.SemaphoreType.DMA,
            pltpu.SemaphoreType.DMA,
        ],
    )
    return f(ne, dst, zeros)


def kernel(node_feat, edge_feat, edge_index,
           ew0, eb0, ew1, eb1, ew2, eb2,
           nw0, nb0, nw1, nb1, nw2, nb2):
    src = edge_index[0].astype(jnp.int32)
    dst = edge_index[1].astype(jnp.int32)
    ps, pd = _node_proj(node_feat, ew0[D:2 * D], ew0[2 * D:])
    gsum = _sc_gather(ps, pd, src, dst)
    new_edge = _edge_mlp(edge_feat, gsum, ew0[:D], eb0, ew1, eb1, ew2, eb2)
    partials = _sc_scatter(new_edge, dst, jnp.zeros((N_PAD, D), _f32))
    new_node = _node_mlp(node_feat, partials, nw0[:D], nw0[D:], nb0,
                         nw1, nb1, nw2, nb2)
    return (new_node, new_edge)


# speedup vs baseline: 1.1195x; 1.1123x over previous
import jax
import jax.numpy as jnp
from jax import lax
from jax.experimental import pallas as pl
from jax.experimental.pallas import tpu as pltpu
from jax.experimental.pallas import tpu_sc as plsc

N_NODES = 10000
N_EDGES = 320000
D = 128

NC = 2
NS = 16
NW = NC * NS
EPW = N_EDGES // NW
CHUNK = 80
NCHUNK = EPW // CHUNK
SCHUNK = 128
NSC = EPW // SCHUNK
TAIL = EPW - NSC * SCHUNK
N_PAD = 10240
ROWS_PER_TILE = N_PAD // NS

_f32 = jnp.float32


def _proj_body(nf_ref, ws_ref, wd_ref, ps_ref, pd_ref):
    nf = nf_ref[...]
    ps_ref[...] = jnp.dot(nf, ws_ref[...], preferred_element_type=_f32)
    pd_ref[...] = jnp.dot(nf, wd_ref[...], preferred_element_type=_f32)


def _node_proj(nf, ws, wd):
    blk = 2000
    return pl.pallas_call(
        _proj_body,
        grid=(N_NODES // blk,),
        in_specs=[
            pl.BlockSpec((blk, D), lambda i: (i, 0)),
            pl.BlockSpec((D, D), lambda i: (0, 0)),
            pl.BlockSpec((D, D), lambda i: (0, 0)),
        ],
        out_specs=[
            pl.BlockSpec((blk, D), lambda i: (i, 0)),
            pl.BlockSpec((blk, D), lambda i: (i, 0)),
        ],
        out_shape=[jax.ShapeDtypeStruct((N_NODES, D), _f32)] * 2,
    )(nf, ws, wd)


def _edge_body(ef_ref, gsum_ref, w0_ref, b0_ref, w1_ref, b1_ref,
               w2_ref, b2_ref, out_ref):
    ef = ef_ref[...]
    h = jnp.dot(ef, w0_ref[...], preferred_element_type=_f32)
    h = jnp.maximum(h + gsum_ref[...] + b0_ref[...], 0.0)
    h = jnp.maximum(
        jnp.dot(h, w1_ref[...], preferred_element_type=_f32) + b1_ref[...], 0.0)
    out_ref[...] = ef + jnp.dot(h, w2_ref[...], preferred_element_type=_f32) \
        + b2_ref[...]


def _edge_mlp(ef, gsum, w0, b0, w1, b1, w2, b2):
    blk = 4000
    wspec = pl.BlockSpec((D, D), lambda i: (0, 0))
    bspec = pl.BlockSpec((1, D), lambda i: (0, 0))
    espec = pl.BlockSpec((blk, D), lambda i: (i, 0))
    return pl.pallas_call(
        _edge_body,
        grid=(N_EDGES // blk,),
        in_specs=[espec, espec, wspec, bspec, wspec, bspec, wspec, bspec],
        out_specs=espec,
        out_shape=jax.ShapeDtypeStruct((N_EDGES, D), _f32),
    )(ef, gsum, w0, b0.reshape(1, D), w1, b1.reshape(1, D), w2,
      b2.reshape(1, D))


def _node_body(nf_ref, p_ref, w0n_ref, w0a_ref, b0_ref, w1_ref, b1_ref,
               w2_ref, b2_ref, out_ref):
    nf = nf_ref[...]
    agg = p_ref[0] + p_ref[1]
    h = jnp.dot(nf, w0n_ref[...], preferred_element_type=_f32) \
        + jnp.dot(agg, w0a_ref[...], preferred_element_type=_f32)
    h = jnp.maximum(h + b0_ref[...], 0.0)
    h = jnp.maximum(
        jnp.dot(h, w1_ref[...], preferred_element_type=_f32) + b1_ref[...], 0.0)
    out_ref[...] = nf + jnp.dot(h, w2_ref[...], preferred_element_type=_f32) \
        + b2_ref[...]


def _node_mlp(nf, partials, w0n, w0a, b0, w1, b1, w2, b2):
    blk = 2000
    wspec = pl.BlockSpec((D, D), lambda i: (0, 0))
    bspec = pl.BlockSpec((1, D), lambda i: (0, 0))
    nspec = pl.BlockSpec((blk, D), lambda i: (i, 0))
    return pl.pallas_call(
        _node_body,
        grid=(N_NODES // blk,),
        in_specs=[
            nspec,
            pl.BlockSpec((NC, blk, D), lambda i: (0, i, 0)),
            wspec, wspec, bspec, wspec, bspec, wspec, bspec,
        ],
        out_specs=nspec,
        out_shape=jax.ShapeDtypeStruct((N_NODES, D), _f32),
    )(nf, partials, w0n, w0a, b0.reshape(1, D), w1, b1.reshape(1, D), w2,
      b2.reshape(1, D))


def _sc_gather_body(ps_hbm, pd_hbm, src_hbm, dst_hbm, gsum_hbm,
                    idx_sv, idx_dv, rows,
                    sem_a0, sem_a1, sem_a2, sem_a3, sem_a4, sem_a5,
                    sem_b0, sem_b1, sem_b2, sem_b3, sem_b4, sem_b5):
    c = lax.axis_index("c")
    s = lax.axis_index("s")
    wid = s * NC + c
    ebase = wid * EPW
    sem_a = (sem_a0, sem_a1, sem_a2, sem_a3, sem_a4, sem_a5)
    sem_b = (sem_b0, sem_b1, sem_b2, sem_b3, sem_b4, sem_b5)

    pltpu.sync_copy(src_hbm.at[pl.ds(ebase, EPW)], idx_sv)
    pltpu.sync_copy(dst_hbm.at[pl.ds(ebase, EPW)], idx_dv)

    def issue_a(j, p):
        ia = idx_sv.at[pl.ds(j * CHUNK, CHUNK)]
        pltpu.async_copy(ps_hbm.at[ia], rows.at[p], sem_a[p])

    def wait_a(p):
        ia = idx_sv.at[pl.ds(0, CHUNK)]
        pltpu.make_async_copy(ps_hbm.at[ia], rows.at[p], sem_a[p]).wait()

    def issue_b(j, p):
        ib = idx_dv.at[pl.ds(j * CHUNK, CHUNK)]
        pltpu.async_copy(pd_hbm.at[ib], rows.at[p], sem_b[p], add=True)

    def wait_b(p):
        ib = idx_dv.at[pl.ds(0, CHUNK)]
        pltpu.make_async_copy(pd_hbm.at[ib], rows.at[p], sem_b[p]).wait()

    def write(j, p):
        base = ebase + j * CHUNK
        pltpu.sync_copy(rows.at[p], gsum_hbm.at[pl.ds(base, CHUNK)])

    def full_step(j, s, s2, guard_refill):
        wait_a(s)
        issue_b(j, s)
        wait_b(s2)
        write(j - 2, s2)
        if guard_refill:
            @pl.when(j + 4 < NCHUNK)
            def _():
                issue_a(j + 4, s2)
        else:
            issue_a(j + 4, s2)

    issue_a(0, 0)
    issue_a(1, 1)
    issue_a(2, 2)
    issue_a(3, 3)
    wait_a(0)
    issue_b(0, 0)
    issue_a(4, 4)
    wait_a(1)
    issue_b(1, 1)
    issue_a(5, 5)

    def sextet(g, carry):
        j0 = 6 * g + 2
        full_step(j0, 2, 0, False)
        full_step(j0 + 1, 3, 1, False)
        full_step(j0 + 2, 4, 2, False)
        full_step(j0 + 3, 5, 3, False)
        full_step(j0 + 4, 0, 4, False)
        full_step(j0 + 5, 1, 5, True)
        return carry

    lax.fori_loop(0, (NCHUNK - 5) // 6, sextet, 0)
    wait_a(2)
    issue_b(NCHUNK - 3, 2)
    wait_b(0)
    write(NCHUNK - 5, 0)
    wait_a(3)
    issue_b(NCHUNK - 2, 3)
    wait_b(1)
    write(NCHUNK - 4, 1)
    wait_a(4)
    issue_b(NCHUNK - 1, 4)
    wait_b(2)
    write(NCHUNK - 3, 2)
    wait_b(3)
    write(NCHUNK - 2, 3)
    wait_b(4)
    write(NCHUNK - 1, 4)


def _sc_gather(ps, pd, src, dst):
    f = pl.kernel(
        _sc_gather_body,
        out_type=jax.ShapeDtypeStruct((N_EDGES, D), _f32),
        mesh=plsc.VectorSubcoreMesh(core_axis_name="c", subcore_axis_name="s",
                                    num_cores=NC, num_subcores=NS),
        scratch_types=[
            pltpu.VMEM((EPW,), jnp.int32),
            pltpu.VMEM((EPW,), jnp.int32),
            pltpu.VMEM((6, CHUNK, D), _f32),
        ] + [pltpu.SemaphoreType.DMA] * 12,
    )
    return f(ps, pd, src, dst)


def _sc_scatter_body(ne_hbm, dst_hbm, zeros_hbm, out_hbm,
                     acc_shared, idx_v, rows_v, sem_l0, sem_l1):
    c = lax.axis_index("c")
    s = lax.axis_index("s")
    wid = s * NC + c
    ebase = wid * EPW
    row0 = s * ROWS_PER_TILE
    sem_l = (sem_l0, sem_l1)
    pltpu.sync_copy(zeros_hbm.at[pl.ds(row0, ROWS_PER_TILE)],
                    acc_shared.at[pl.ds(row0, ROWS_PER_TILE)])
    pltpu.sync_copy(dst_hbm.at[pl.ds(ebase, EPW)], idx_v)
    plsc.subcore_barrier()

    def load(j, p):
        base = ebase + j * SCHUNK
        pltpu.async_copy(ne_hbm.at[pl.ds(base, SCHUNK)], rows_v.at[p],
                         sem_l[p])

    def wait_load(p):
        pltpu.make_async_copy(ne_hbm.at[pl.ds(0, SCHUNK)], rows_v.at[p],
                              sem_l[p]).wait()

    def scat(j, p):
        idx = idx_v.at[pl.ds(j * SCHUNK, SCHUNK)]
        pltpu.sync_copy(rows_v.at[p], acc_shared.at[idx], add=True)

    load(0, 0)
    load(1, 1)

    def pair(jj, carry):
        e = 2 * jj
        o = e + 1
        wait_load(0)
        scat(e, 0)

        @pl.when(e + 2 < NSC)
        def _():
            load(e + 2, 0)

        wait_load(1)
        scat(o, 1)

        @pl.when(o + 2 < NSC)
        def _():
            load(o + 2, 1)

        return carry

    lax.fori_loop(0, NSC // 2, pair, 0)
    pltpu.sync_copy(ne_hbm.at[pl.ds(ebase + NSC * SCHUNK, TAIL)],
                    rows_v.at[0, pl.ds(0, TAIL)])
    pltpu.sync_copy(rows_v.at[0, pl.ds(0, TAIL)],
                    acc_shared.at[idx_v.at[pl.ds(NSC * SCHUNK, TAIL)]],
                    add=True)

    plsc.subcore_barrier()
    pltpu.sync_copy(acc_shared.at[pl.ds(row0, ROWS_PER_TILE)],
                    out_hbm.at[c, pl.ds(row0, ROWS_PER_TILE)])


def _sc_scatter(ne, dst, zeros):
    f = pl.kernel(
        _sc_scatter_body,
        out_type=jax.ShapeDtypeStruct((NC, N_PAD, D), _f32),
        mesh=plsc.VectorSubcoreMesh(core_axis_name="c", subcore_axis_name="s",
                                    num_cores=NC, num_subcores=NS),
        scratch_types=[
            pltpu.VMEM_SHARED((N_PAD, D), _f32),
            pltpu.VMEM((EPW,), jnp.int32),
            pltpu.VMEM((2, SCHUNK, D), _f32),
            pltpu.SemaphoreType.DMA,
            pltpu.SemaphoreType.DMA,
        ],
    )
    return f(ne, dst, zeros)


def kernel(node_feat, edge_feat, edge_index,
           ew0, eb0, ew1, eb1, ew2, eb2,
           nw0, nb0, nw1, nb1, nw2, nb2):
    src = edge_index[0].astype(jnp.int32)
    dst = edge_index[1].astype(jnp.int32)
    ps, pd = _node_proj(node_feat, ew0[D:2 * D], ew0[2 * D:])
    gsum = _sc_gather(ps, pd, src, dst)
    new_edge = _edge_mlp(edge_feat, gsum, ew0[:D], eb0, ew1, eb1, ew2, eb2)
    partials = _sc_scatter(new_edge, dst, jnp.zeros((N_PAD, D), _f32))
    new_node = _node_mlp(node_feat, partials, nw0[:D], nw0[D:], nb0,
                         nw1, nb1, nw2, nb2)
    return (new_node, new_edge)


# speedup vs baseline: 1.1530x; 1.0299x over previous
import jax
import jax.numpy as jnp
from jax import lax
from jax.experimental import pallas as pl
from jax.experimental.pallas import tpu as pltpu
from jax.experimental.pallas import tpu_sc as plsc

N_NODES = 10000
N_EDGES = 320000
D = 128

NC = 2
NS = 16
NW = NC * NS
EPW = N_EDGES // NW
CHUNK = 80
NCHUNK = EPW // CHUNK
SCHUNK = 128
NSC = EPW // SCHUNK
TAIL = EPW - NSC * SCHUNK
N_PAD = 10240
ROWS_PER_TILE = N_PAD // NS

_f32 = jnp.float32


def _proj_body(nf_ref, ws_ref, wd_ref, ps_ref, pd_ref):
    nf = nf_ref[...]
    ps_ref[...] = jnp.dot(nf, ws_ref[...], preferred_element_type=_f32)
    pd_ref[...] = jnp.dot(nf, wd_ref[...], preferred_element_type=_f32)


def _node_proj(nf, ws, wd):
    blk = 5000
    return pl.pallas_call(
        _proj_body,
        grid=(N_NODES // blk,),
        in_specs=[
            pl.BlockSpec((blk, D), lambda i: (i, 0)),
            pl.BlockSpec((D, D), lambda i: (0, 0)),
            pl.BlockSpec((D, D), lambda i: (0, 0)),
        ],
        out_specs=[
            pl.BlockSpec((blk, D), lambda i: (i, 0)),
            pl.BlockSpec((blk, D), lambda i: (i, 0)),
        ],
        out_shape=[jax.ShapeDtypeStruct((N_NODES, D), _f32)] * 2,
    )(nf, ws, wd)


def _edge_body(ef_ref, gsum_ref, w0_ref, b0_ref, w1_ref, b1_ref,
               w2_ref, b2_ref, out_ref):
    ef = ef_ref[...]
    h = jnp.dot(ef, w0_ref[...], preferred_element_type=_f32)
    h = jnp.maximum(h + gsum_ref[...] + b0_ref[...], 0.0)
    h = jnp.maximum(
        jnp.dot(h, w1_ref[...], preferred_element_type=_f32) + b1_ref[...], 0.0)
    out_ref[...] = ef + jnp.dot(h, w2_ref[...], preferred_element_type=_f32) \
        + b2_ref[...]


def _edge_mlp(ef, gsum, w0, b0, w1, b1, w2, b2):
    blk = 8000
    wspec = pl.BlockSpec((D, D), lambda i: (0, 0))
    bspec = pl.BlockSpec((1, D), lambda i: (0, 0))
    espec = pl.BlockSpec((blk, D), lambda i: (i, 0))
    return pl.pallas_call(
        _edge_body,
        grid=(N_EDGES // blk,),
        in_specs=[espec, espec, wspec, bspec, wspec, bspec, wspec, bspec],
        out_specs=espec,
        out_shape=jax.ShapeDtypeStruct((N_EDGES, D), _f32),
    )(ef, gsum, w0, b0.reshape(1, D), w1, b1.reshape(1, D), w2,
      b2.reshape(1, D))


def _node_body(nf_ref, p_ref, w0n_ref, w0a_ref, b0_ref, w1_ref, b1_ref,
               w2_ref, b2_ref, out_ref):
    nf = nf_ref[...]
    agg = p_ref[0] + p_ref[1]
    h = jnp.dot(nf, w0n_ref[...], preferred_element_type=_f32) \
        + jnp.dot(agg, w0a_ref[...], preferred_element_type=_f32)
    h = jnp.maximum(h + b0_ref[...], 0.0)
    h = jnp.maximum(
        jnp.dot(h, w1_ref[...], preferred_element_type=_f32) + b1_ref[...], 0.0)
    out_ref[...] = nf + jnp.dot(h, w2_ref[...], preferred_element_type=_f32) \
        + b2_ref[...]


def _node_mlp(nf, partials, w0n, w0a, b0, w1, b1, w2, b2):
    blk = 5000
    wspec = pl.BlockSpec((D, D), lambda i: (0, 0))
    bspec = pl.BlockSpec((1, D), lambda i: (0, 0))
    nspec = pl.BlockSpec((blk, D), lambda i: (i, 0))
    return pl.pallas_call(
        _node_body,
        grid=(N_NODES // blk,),
        in_specs=[
            nspec,
            pl.BlockSpec((NC, blk, D), lambda i: (0, i, 0)),
            wspec, wspec, bspec, wspec, bspec, wspec, bspec,
        ],
        out_specs=nspec,
        out_shape=jax.ShapeDtypeStruct((N_NODES, D), _f32),
    )(nf, partials, w0n, w0a, b0.reshape(1, D), w1, b1.reshape(1, D), w2,
      b2.reshape(1, D))


def _sc_gather_body(ps_hbm, pd_hbm, src_hbm, dst_hbm, gsum_hbm,
                    idx_sv, idx_dv, rows,
                    sem_a0, sem_a1, sem_a2, sem_a3, sem_a4, sem_a5,
                    sem_b0, sem_b1, sem_b2, sem_b3, sem_b4, sem_b5):
    c = lax.axis_index("c")
    s = lax.axis_index("s")
    wid = s * NC + c
    ebase = wid * EPW
    sem_a = (sem_a0, sem_a1, sem_a2, sem_a3, sem_a4, sem_a5)
    sem_b = (sem_b0, sem_b1, sem_b2, sem_b3, sem_b4, sem_b5)

    pltpu.sync_copy(src_hbm.at[pl.ds(ebase, EPW)], idx_sv)
    pltpu.sync_copy(dst_hbm.at[pl.ds(ebase, EPW)], idx_dv)

    def issue_a(j, p):
        ia = idx_sv.at[pl.ds(j * CHUNK, CHUNK)]
        pltpu.async_copy(ps_hbm.at[ia], rows.at[p], sem_a[p])

    def wait_a(p):
        ia = idx_sv.at[pl.ds(0, CHUNK)]
        pltpu.make_async_copy(ps_hbm.at[ia], rows.at[p], sem_a[p]).wait()

    def issue_b(j, p):
        ib = idx_dv.at[pl.ds(j * CHUNK, CHUNK)]
        pltpu.async_copy(pd_hbm.at[ib], rows.at[p], sem_b[p], add=True)

    def wait_b(p):
        ib = idx_dv.at[pl.ds(0, CHUNK)]
        pltpu.make_async_copy(pd_hbm.at[ib], rows.at[p], sem_b[p]).wait()

    def write(j, p):
        base = ebase + j * CHUNK
        pltpu.sync_copy(rows.at[p], gsum_hbm.at[pl.ds(base, CHUNK)])

    def full_step(j, s, s2, guard_refill):
        wait_a(s)
        issue_b(j, s)
        wait_b(s2)
        write(j - 2, s2)
        if guard_refill:
            @pl.when(j + 4 < NCHUNK)
            def _():
                issue_a(j + 4, s2)
        else:
            issue_a(j + 4, s2)

    issue_a(0, 0)
    issue_a(1, 1)
    issue_a(2, 2)
    issue_a(3, 3)
    wait_a(0)
    issue_b(0, 0)
    issue_a(4, 4)
    wait_a(1)
    issue_b(1, 1)
    issue_a(5, 5)

    def sextet(g, carry):
        j0 = 6 * g + 2
        full_step(j0, 2, 0, False)
        full_step(j0 + 1, 3, 1, False)
        full_step(j0 + 2, 4, 2, False)
        full_step(j0 + 3, 5, 3, False)
        full_step(j0 + 4, 0, 4, False)
        full_step(j0 + 5, 1, 5, True)
        return carry

    lax.fori_loop(0, (NCHUNK - 5) // 6, sextet, 0)
    wait_a(2)
    issue_b(NCHUNK - 3, 2)
    wait_b(0)
    write(NCHUNK - 5, 0)
    wait_a(3)
    issue_b(NCHUNK - 2, 3)
    wait_b(1)
    write(NCHUNK - 4, 1)
    wait_a(4)
    issue_b(NCHUNK - 1, 4)
    wait_b(2)
    write(NCHUNK - 3, 2)
    wait_b(3)
    write(NCHUNK - 2, 3)
    wait_b(4)
    write(NCHUNK - 1, 4)


def _sc_gather(ps, pd, src, dst):
    f = pl.kernel(
        _sc_gather_body,
        out_type=jax.ShapeDtypeStruct((N_EDGES, D), _f32),
        mesh=plsc.VectorSubcoreMesh(core_axis_name="c", subcore_axis_name="s",
                                    num_cores=NC, num_subcores=NS),
        scratch_types=[
            pltpu.VMEM((EPW,), jnp.int32),
            pltpu.VMEM((EPW,), jnp.int32),
            pltpu.VMEM((6, CHUNK, D), _f32),
        ] + [pltpu.SemaphoreType.DMA] * 12,
    )
    return f(ps, pd, src, dst)


def _sc_scatter_body(ne_hbm, dst_hbm, zeros_hbm, out_hbm,
                     acc_shared, idx_v, rows_v, sem_l0, sem_l1):
    c = lax.axis_index("c")
    s = lax.axis_index("s")
    wid = s * NC + c
    ebase = wid * EPW
    row0 = s * ROWS_PER_TILE
    sem_l = (sem_l0, sem_l1)
    pltpu.sync_copy(zeros_hbm.at[pl.ds(row0, ROWS_PER_TILE)],
                    acc_shared.at[pl.ds(row0, ROWS_PER_TILE)])
    pltpu.sync_copy(dst_hbm.at[pl.ds(ebase, EPW)], idx_v)
    plsc.subcore_barrier()

    def load(j, p):
        base = ebase + j * SCHUNK
        pltpu.async_copy(ne_hbm.at[pl.ds(base, SCHUNK)], rows_v.at[p],
                         sem_l[p])

    def wait_load(p):
        pltpu.make_async_copy(ne_hbm.at[pl.ds(0, SCHUNK)], rows_v.at[p],
                              sem_l[p]).wait()

    def scat(j, p):
        idx = idx_v.at[pl.ds(j * SCHUNK, SCHUNK)]
        pltpu.sync_copy(rows_v.at[p], acc_shared.at[idx], add=True)

    load(0, 0)
    load(1, 1)

    def pair(jj, carry):
        e = 2 * jj
        o = e + 1
        wait_load(0)
        scat(e, 0)

        @pl.when(e + 2 < NSC)
        def _():
            load(e + 2, 0)

        wait_load(1)
        scat(o, 1)

        @pl.when(o + 2 < NSC)
        def _():
            load(o + 2, 1)

        return carry

    lax.fori_loop(0, NSC // 2, pair, 0)
    pltpu.sync_copy(ne_hbm.at[pl.ds(ebase + NSC * SCHUNK, TAIL)],
                    rows_v.at[0, pl.ds(0, TAIL)])
    pltpu.sync_copy(rows_v.at[0, pl.ds(0, TAIL)],
                    acc_shared.at[idx_v.at[pl.ds(NSC * SCHUNK, TAIL)]],
                    add=True)

    plsc.subcore_barrier()
    pltpu.sync_copy(acc_shared.at[pl.ds(row0, ROWS_PER_TILE)],
                    out_hbm.at[c, pl.ds(row0, ROWS_PER_TILE)])


def _sc_scatter(ne, dst, zeros):
    f = pl.kernel(
        _sc_scatter_body,
        out_type=jax.ShapeDtypeStruct((NC, N_PAD, D), _f32),
        mesh=plsc.VectorSubcoreMesh(core_axis_name="c", subcore_axis_name="s",
                                    num_cores=NC, num_subcores=NS),
        scratch_types=[
            pltpu.VMEM_SHARED((N_PAD, D), _f32),
            pltpu.VMEM((EPW,), jnp.int32),
            pltpu.VMEM((2, SCHUNK, D), _f32),
            pltpu.SemaphoreType.DMA,
            pltpu.SemaphoreType.DMA,
        ],
    )
    return f(ne, dst, zeros)


def kernel(node_feat, edge_feat, edge_index,
           ew0, eb0, ew1, eb1, ew2, eb2,
           nw0, nb0, nw1, nb1, nw2, nb2):
    src = edge_index[0].astype(jnp.int32)
    dst = edge_index[1].astype(jnp.int32)
    ps, pd = _node_proj(node_feat, ew0[D:2 * D], ew0[2 * D:])
    gsum = _sc_gather(ps, pd, src, dst)
    new_edge = _edge_mlp(edge_feat, gsum, ew0[:D], eb0, ew1, eb1, ew2, eb2)
    partials = _sc_scatter(new_edge, dst, jnp.zeros((N_PAD, D), _f32))
    new_node = _node_mlp(node_feat, partials, nw0[:D], nw0[D:], nb0,
                         nw1, nb1, nw2, nb2)
    return (new_node, new_edge)


# speedup vs baseline: 1.1674x; 1.0125x over previous
import jax
import jax.numpy as jnp
from jax import lax
from jax.experimental import pallas as pl
from jax.experimental.pallas import tpu as pltpu
from jax.experimental.pallas import tpu_sc as plsc

N_NODES = 10000
N_EDGES = 320000
D = 128

NC = 2
NS = 16
NW = NC * NS
EPW = N_EDGES // NW
CHUNK = 80
NCHUNK = EPW // CHUNK
SCHUNK = 128
NSC = EPW // SCHUNK
TAIL = EPW - NSC * SCHUNK
N_PAD = 10240
ROWS_PER_TILE = N_PAD // NS

_f32 = jnp.float32


def _proj_body(nf_ref, ws_ref, wd_ref, ps_ref, pd_ref):
    nf = nf_ref[...]
    ps_ref[...] = jnp.dot(nf, ws_ref[...], preferred_element_type=_f32)
    pd_ref[...] = jnp.dot(nf, wd_ref[...], preferred_element_type=_f32)


def _node_proj(nf, ws, wd):
    blk = 10000
    return pl.pallas_call(
        _proj_body,
        grid=(N_NODES // blk,),
        in_specs=[
            pl.BlockSpec((blk, D), lambda i: (i, 0)),
            pl.BlockSpec((D, D), lambda i: (0, 0)),
            pl.BlockSpec((D, D), lambda i: (0, 0)),
        ],
        out_specs=[
            pl.BlockSpec((blk, D), lambda i: (i, 0)),
            pl.BlockSpec((blk, D), lambda i: (i, 0)),
        ],
        out_shape=[jax.ShapeDtypeStruct((N_NODES, D), _f32)] * 2,
    )(nf, ws, wd)


def _edge_body(ef_ref, gsum_ref, w0_ref, b0_ref, w1_ref, b1_ref,
               w2_ref, b2_ref, out_ref):
    ef = ef_ref[...]
    h = jnp.dot(ef, w0_ref[...], preferred_element_type=_f32)
    h = jnp.maximum(h + gsum_ref[...] + b0_ref[...], 0.0)
    h = jnp.maximum(
        jnp.dot(h, w1_ref[...], preferred_element_type=_f32) + b1_ref[...], 0.0)
    out_ref[...] = ef + jnp.dot(h, w2_ref[...], preferred_element_type=_f32) \
        + b2_ref[...]


def _edge_mlp(ef, gsum, w0, b0, w1, b1, w2, b2):
    blk = 16000
    wspec = pl.BlockSpec((D, D), lambda i: (0, 0))
    bspec = pl.BlockSpec((1, D), lambda i: (0, 0))
    espec = pl.BlockSpec((blk, D), lambda i: (i, 0))
    return pl.pallas_call(
        _edge_body,
        grid=(N_EDGES // blk,),
        in_specs=[espec, espec, wspec, bspec, wspec, bspec, wspec, bspec],
        out_specs=espec,
        out_shape=jax.ShapeDtypeStruct((N_EDGES, D), _f32),
    )(ef, gsum, w0, b0.reshape(1, D), w1, b1.reshape(1, D), w2,
      b2.reshape(1, D))


def _node_body(nf_ref, p_ref, w0n_ref, w0a_ref, b0_ref, w1_ref, b1_ref,
               w2_ref, b2_ref, out_ref):
    nf = nf_ref[...]
    agg = p_ref[0] + p_ref[1]
    h = jnp.dot(nf, w0n_ref[...], preferred_element_type=_f32) \
        + jnp.dot(agg, w0a_ref[...], preferred_element_type=_f32)
    h = jnp.maximum(h + b0_ref[...], 0.0)
    h = jnp.maximum(
        jnp.dot(h, w1_ref[...], preferred_element_type=_f32) + b1_ref[...], 0.0)
    out_ref[...] = nf + jnp.dot(h, w2_ref[...], preferred_element_type=_f32) \
        + b2_ref[...]


def _node_mlp(nf, partials, w0n, w0a, b0, w1, b1, w2, b2):
    blk = 10000
    wspec = pl.BlockSpec((D, D), lambda i: (0, 0))
    bspec = pl.BlockSpec((1, D), lambda i: (0, 0))
    nspec = pl.BlockSpec((blk, D), lambda i: (i, 0))
    return pl.pallas_call(
        _node_body,
        grid=(N_NODES // blk,),
        in_specs=[
            nspec,
            pl.BlockSpec((NC, blk, D), lambda i: (0, i, 0)),
            wspec, wspec, bspec, wspec, bspec, wspec, bspec,
        ],
        out_specs=nspec,
        out_shape=jax.ShapeDtypeStruct((N_NODES, D), _f32),
    )(nf, partials, w0n, w0a, b0.reshape(1, D), w1, b1.reshape(1, D), w2,
      b2.reshape(1, D))


def _sc_gather_body(ps_hbm, pd_hbm, src_hbm, dst_hbm, gsum_hbm,
                    idx_sv, idx_dv, rows,
                    sem_a0, sem_a1, sem_a2, sem_a3, sem_a4, sem_a5,
                    sem_b0, sem_b1, sem_b2, sem_b3, sem_b4, sem_b5):
    c = lax.axis_index("c")
    s = lax.axis_index("s")
    wid = s * NC + c
    ebase = wid * EPW
    sem_a = (sem_a0, sem_a1, sem_a2, sem_a3, sem_a4, sem_a5)
    sem_b = (sem_b0, sem_b1, sem_b2, sem_b3, sem_b4, sem_b5)

    pltpu.sync_copy(src_hbm.at[pl.ds(ebase, EPW)], idx_sv)
    pltpu.sync_copy(dst_hbm.at[pl.ds(ebase, EPW)], idx_dv)

    def issue_a(j, p):
        ia = idx_sv.at[pl.ds(j * CHUNK, CHUNK)]
        pltpu.async_copy(ps_hbm.at[ia], rows.at[p], sem_a[p])

    def wait_a(p):
        ia = idx_sv.at[pl.ds(0, CHUNK)]
        pltpu.make_async_copy(ps_hbm.at[ia], rows.at[p], sem_a[p]).wait()

    def issue_b(j, p):
        ib = idx_dv.at[pl.ds(j * CHUNK, CHUNK)]
        pltpu.async_copy(pd_hbm.at[ib], rows.at[p], sem_b[p], add=True)

    def wait_b(p):
        ib = idx_dv.at[pl.ds(0, CHUNK)]
        pltpu.make_async_copy(pd_hbm.at[ib], rows.at[p], sem_b[p]).wait()

    def write(j, p):
        base = ebase + j * CHUNK
        pltpu.sync_copy(rows.at[p], gsum_hbm.at[pl.ds(base, CHUNK)])

    def full_step(j, s, s2, guard_refill):
        wait_a(s)
        issue_b(j, s)
        wait_b(s2)
        write(j - 2, s2)
        if guard_refill:
            @pl.when(j + 4 < NCHUNK)
            def _():
                issue_a(j + 4, s2)
        else:
            issue_a(j + 4, s2)

    issue_a(0, 0)
    issue_a(1, 1)
    issue_a(2, 2)
    issue_a(3, 3)
    wait_a(0)
    issue_b(0, 0)
    issue_a(4, 4)
    wait_a(1)
    issue_b(1, 1)
    issue_a(5, 5)

    def sextet(g, carry):
        j0 = 6 * g + 2
        full_step(j0, 2, 0, False)
        full_step(j0 + 1, 3, 1, False)
        full_step(j0 + 2, 4, 2, False)
        full_step(j0 + 3, 5, 3, False)
        full_step(j0 + 4, 0, 4, False)
        full_step(j0 + 5, 1, 5, True)
        return carry

    lax.fori_loop(0, (NCHUNK - 5) // 6, sextet, 0)
    wait_a(2)
    issue_b(NCHUNK - 3, 2)
    wait_b(0)
    write(NCHUNK - 5, 0)
    wait_a(3)
    issue_b(NCHUNK - 2, 3)
    wait_b(1)
    write(NCHUNK - 4, 1)
    wait_a(4)
    issue_b(NCHUNK - 1, 4)
    wait_b(2)
    write(NCHUNK - 3, 2)
    wait_b(3)
    write(NCHUNK - 2, 3)
    wait_b(4)
    write(NCHUNK - 1, 4)


def _sc_gather(ps, pd, src, dst):
    f = pl.kernel(
        _sc_gather_body,
        out_type=jax.ShapeDtypeStruct((N_EDGES, D), _f32),
        mesh=plsc.VectorSubcoreMesh(core_axis_name="c", subcore_axis_name="s",
                                    num_cores=NC, num_subcores=NS),
        scratch_types=[
            pltpu.VMEM((EPW,), jnp.int32),
            pltpu.VMEM((EPW,), jnp.int32),
            pltpu.VMEM((6, CHUNK, D), _f32),
        ] + [pltpu.SemaphoreType.DMA] * 12,
    )
    return f(ps, pd, src, dst)


def _sc_scatter_body(ne_hbm, dst_hbm, zeros_hbm, out_hbm,
                     acc_shared, idx_v, rows_v, sem_l0, sem_l1):
    c = lax.axis_index("c")
    s = lax.axis_index("s")
    wid = s * NC + c
    ebase = wid * EPW
    row0 = s * ROWS_PER_TILE
    sem_l = (sem_l0, sem_l1)
    pltpu.sync_copy(zeros_hbm.at[pl.ds(row0, ROWS_PER_TILE)],
                    acc_shared.at[pl.ds(row0, ROWS_PER_TILE)])
    pltpu.sync_copy(dst_hbm.at[pl.ds(ebase, EPW)], idx_v)
    plsc.subcore_barrier()

    def load(j, p):
        base = ebase + j * SCHUNK
        pltpu.async_copy(ne_hbm.at[pl.ds(base, SCHUNK)], rows_v.at[p],
                         sem_l[p])

    def wait_load(p):
        pltpu.make_async_copy(ne_hbm.at[pl.ds(0, SCHUNK)], rows_v.at[p],
                              sem_l[p]).wait()

    def scat(j, p):
        idx = idx_v.at[pl.ds(j * SCHUNK, SCHUNK)]
        pltpu.sync_copy(rows_v.at[p], acc_shared.at[idx], add=True)

    load(0, 0)
    load(1, 1)

    def pair(jj, carry):
        e = 2 * jj
        o = e + 1
        wait_load(0)
        scat(e, 0)

        @pl.when(e + 2 < NSC)
        def _():
            load(e + 2, 0)

        wait_load(1)
        scat(o, 1)

        @pl.when(o + 2 < NSC)
        def _():
            load(o + 2, 1)

        return carry

    lax.fori_loop(0, NSC // 2, pair, 0)
    pltpu.sync_copy(ne_hbm.at[pl.ds(ebase + NSC * SCHUNK, TAIL)],
                    rows_v.at[0, pl.ds(0, TAIL)])
    pltpu.sync_copy(rows_v.at[0, pl.ds(0, TAIL)],
                    acc_shared.at[idx_v.at[pl.ds(NSC * SCHUNK, TAIL)]],
                    add=True)

    plsc.subcore_barrier()
    pltpu.sync_copy(acc_shared.at[pl.ds(row0, ROWS_PER_TILE)],
                    out_hbm.at[c, pl.ds(row0, ROWS_PER_TILE)])


def _sc_scatter(ne, dst, zeros):
    f = pl.kernel(
        _sc_scatter_body,
        out_type=jax.ShapeDtypeStruct((NC, N_PAD, D), _f32),
        mesh=plsc.VectorSubcoreMesh(core_axis_name="c", subcore_axis_name="s",
                                    num_cores=NC, num_subcores=NS),
        scratch_types=[
            pltpu.VMEM_SHARED((N_PAD, D), _f32),
            pltpu.VMEM((EPW,), jnp.int32),
            pltpu.VMEM((2, SCHUNK, D), _f32),
            pltpu.SemaphoreType.DMA,
            pltpu.SemaphoreType.DMA,
        ],
    )
    return f(ne, dst, zeros)


def kernel(node_feat, edge_feat, edge_index,
           ew0, eb0, ew1, eb1, ew2, eb2,
           nw0, nb0, nw1, nb1, nw2, nb2):
    src = edge_index[0].astype(jnp.int32)
    dst = edge_index[1].astype(jnp.int32)
    ps, pd = _node_proj(node_feat, ew0[D:2 * D], ew0[2 * D:])
    gsum = _sc_gather(ps, pd, src, dst)
    new_edge = _edge_mlp(edge_feat, gsum, ew0[:D], eb0, ew1, eb1, ew2, eb2)
    partials = _sc_scatter(new_edge, dst, jnp.zeros((N_PAD, D), _f32))
    new_node = _node_mlp(node_feat, partials, nw0[:D], nw0[D:], nb0,
                         nw1, nb1, nw2, nb2)
    return (new_node, new_edge)
